# baseline (speedup 1.0000x reference)
import functools

import jax
import jax.numpy as jnp
from jax import lax
from jax.experimental import pallas as pl
from jax.experimental.pallas import tpu as pltpu
from jax.experimental.pallas import tpu_sc as plsc


def _gather_kernel(F, CS, num_chunks, NC, NW,
                   idx_hbm, tok_hbm, inter_hbm, idx_v, rows_v,
                   sem_idx, sem_g, sem_out):
    w = lax.axis_index("s") * NC + lax.axis_index("c")
    C = F * CS
    n_mine = (num_chunks - 1 - w) // NW + 1

    def fire_idx(k, b):
        pltpu.async_copy(idx_hbm.at[:, pl.ds((w + k * NW) * CS, CS)],
                         idx_v.at[pl.ds(b * F, F)], sem_idx)

    def drain_idx(b):
        pltpu.make_async_copy(idx_hbm.at[:, pl.ds(0, CS)],
                              idx_v.at[pl.ds(b * F, F)], sem_idx).wait()

    def fire_gathers(k, b):
        for f in range(F):
            pltpu.async_copy(tok_hbm.at[idx_v.at[b * F + f]],
                             rows_v.at[pl.ds(b * C + f * CS, CS)], sem_g)

    def drain_gathers(b):
        for f in range(F):
            pltpu.make_async_copy(tok_hbm.at[idx_v.at[b * F + f]],
                                  rows_v.at[pl.ds(b * C + f * CS, CS)],
                                  sem_g).wait()

    def fire_out(k, b):
        pltpu.async_copy(rows_v.at[pl.ds(b * C, C)],
                         inter_hbm.at[pl.ds((w + k * NW) * C, C)], sem_out)

    def drain_out(b):
        pltpu.make_async_copy(rows_v.at[pl.ds(b * C, C)],
                              inter_hbm.at[pl.ds(0, C)], sem_out).wait()

    pltpu.sync_copy(idx_hbm.at[:, pl.ds(w * CS, CS)],
                    idx_v.at[pl.ds(0, F)])
    fire_gathers(0, 0)

    def chunk_body(k, carry):
        b = k & 1
        drain_gathers(b)

        @pl.when(k >= 1)
        def _():
            drain_out(1 - b)

        fire_out(k, b)

        @pl.when(k + 1 < n_mine)
        def _():
            pltpu.sync_copy(
                idx_hbm.at[:, pl.ds((w + (k + 1) * NW) * CS, CS)],
                idx_v.at[pl.ds((1 - b) * F, F)])
            fire_gathers(k + 1, 1 - b)

        return carry

    lax.fori_loop(0, n_mine, chunk_body, 0)
    drain_out(0)


def _addpos_kernel(F, D, CS, num_chunks, NC, NW,
                   in_hbm, pos_hbm, out_hbm, rows_v, buf_v, pos_v,
                   sem_in, sem_out):
    w = lax.axis_index("s") * NC + lax.axis_index("c")
    R = F * CS * D // 128
    FD = F * D
    iota_v = lax.broadcasted_iota(jnp.int32, (D,), 0)
    row_c = lax.shift_right_logical(iota_v, 3)
    dvecs = [(iota_v + j) & 15 for j in range(D)]
    lanevs = [(iota_v & 7) * D + dvecs[j] for j in range(D)]
    pltpu.sync_copy(pos_hbm, pos_v)
    n_mine = (num_chunks - 1 - w) // NW + 1

    pltpu.async_copy(in_hbm.at[pl.ds(w * R, R)],
                     rows_v.at[pl.ds(0, R)], sem_in)

    def chunk_body(k, carry):
        b = k & 1
        sb = w + k * NW
        pltpu.make_async_copy(in_hbm.at[pl.ds(0, R)],
                              rows_v.at[pl.ds(b * R, R)], sem_in).wait()

        @pl.when(k + 1 < n_mine)
        def _():
            pltpu.async_copy(in_hbm.at[pl.ds((sb + NW) * R, R)],
                             rows_v.at[pl.ds((1 - b) * R, R)], sem_in)

        @pl.when(k >= 1)
        def _():
            pltpu.make_async_copy(buf_v.at[pl.ds(0, FD)],
                                  out_hbm.at[:, pl.ds(0, CS)],
                                  sem_out).wait()

        rbase = b * R
        bbase = b * FD

        @functools.partial(plsc.parallel_loop, 0, F, unroll=4)
        def f_body(f):
            pv = pos_v[f >> 3, pl.ds((f & 7) * D, D)]
            rowvs = [row_c + (rbase + f * D + 2 * kk)
                     for kk in range(CS // D)]
            fD = bbase + f * D
            for kk in range(CS // D):
                colv = iota_v + (kk * D)
                for j in range(D):
                    vals = plsc.load_gather(rows_v, [rowvs[kk], lanevs[j]])
                    plsc.store_scatter(buf_v, [dvecs[j] + fD, colv],
                                       vals + pv)
        pltpu.async_copy(buf_v.at[pl.ds(bbase, FD)],
                         out_hbm.at[:, pl.ds(sb * CS, CS)], sem_out)
        return carry

    lax.fori_loop(0, n_mine, chunk_body, 0)
    pltpu.make_async_copy(buf_v.at[pl.ds(0, FD)],
                          out_hbm.at[:, pl.ds(0, CS)], sem_out).wait()


def kernel(inputs, token_table, pos_table):
    S, F = inputs.shape
    V, D = token_table.shape
    assert D == 16 and F <= 16
    CS = 128
    num_chunks = S // CS
    s_main = num_chunks * CS
    C = F * CS

    info = plsc.get_sparse_core_info()
    NC, NS = info.num_cores, info.num_subcores
    NW = NC * NS

    pos_block = pos_table[S:, F:]
    pos_flat = jnp.broadcast_to(pos_block, (F, D)).reshape(-1)
    pos2 = jnp.pad(pos_flat, (0, 256 - F * D)).reshape(2, 128)

    idx_t = inputs.T

    mesh = plsc.VectorSubcoreMesh(core_axis_name="c", subcore_axis_name="s")

    gather_body = functools.partial(_gather_kernel, F, CS, num_chunks, NC, NW)
    inter = pl.kernel(
        gather_body,
        mesh=mesh,
        compiler_params=pltpu.CompilerParams(use_tc_tiling_on_sc=False,
                                             needs_layout_passes=False),
        out_type=jax.ShapeDtypeStruct((num_chunks * C, D), jnp.float32),
        scratch_types=[
            pltpu.VMEM((2 * F, CS), jnp.int32),
            pltpu.VMEM((2 * C, D), jnp.float32),
            pltpu.SemaphoreType.DMA,
            pltpu.SemaphoreType.DMA,
            pltpu.SemaphoreType.DMA,
        ],
    )(idx_t, token_table)

    in128 = inter.reshape(num_chunks * C * D // 128, 128)

    addpos_body = functools.partial(_addpos_kernel, F, D, CS, num_chunks,
                                    NC, NW)
    out_t = pl.kernel(
        addpos_body,
        mesh=mesh,
        compiler_params=pltpu.CompilerParams(needs_layout_passes=False),
        out_type=jax.ShapeDtypeStruct((F * D, S), jnp.float32),
        scratch_types=[
            pltpu.VMEM((2 * C * D // 128, 128), jnp.float32),
            pltpu.VMEM((2 * F * D, CS), jnp.float32),
            pltpu.VMEM((2, 128), jnp.float32),
            pltpu.SemaphoreType.DMA,
            pltpu.SemaphoreType.DMA,
        ],
    )(in128, pos2)

    out = out_t.reshape(F, D, S).transpose(2, 0, 1)
    if s_main < S:
        tail = jnp.take(token_table, inputs[s_main:], axis=0)
        tail = tail + jnp.broadcast_to(pos_block, (F, D))[None]
        out = lax.dynamic_update_slice(out, tail, (s_main, 0, 0))
    return out


# speedup vs baseline: 1.0016x; 1.0016x over previous
import functools

import jax
import jax.numpy as jnp
from jax import lax
from jax.experimental import pallas as pl
from jax.experimental.pallas import tpu as pltpu
from jax.experimental.pallas import tpu_sc as plsc


def _gather_kernel(F, CS, Q, num_chunks, NC, NW,
                   idx_hbm, tok_hbm, inter_hbm, idx_v, rows_v,
                   sem_g, sem_out):
    w = lax.axis_index("s") * NC + lax.axis_index("c")
    C = F * CS
    base_cnt = num_chunks // NW
    rem = num_chunks - base_cnt * NW
    start = w * base_cnt + jnp.minimum(w, rem)
    cnt = base_cnt + jnp.where(w < rem, 1, 0)
    n_groups = (cnt + Q - 1) // Q

    def fire_gathers(k, c0f, b):
        for f in range(F):
            pltpu.async_copy(
                tok_hbm.at[idx_v.at[f, pl.ds((k - c0f) * CS, CS)]],
                rows_v.at[pl.ds(b * C + f * CS, CS)], sem_g)

    def drain_gathers(c0f, b):
        for f in range(F):
            pltpu.make_async_copy(
                tok_hbm.at[idx_v.at[f, pl.ds(0, CS)]],
                rows_v.at[pl.ds(b * C + f * CS, CS)], sem_g).wait()

    def fire_out(k, b):
        pltpu.async_copy(rows_v.at[pl.ds(b * C, C)],
                         inter_hbm.at[pl.ds(k * C, C)], sem_out)

    def drain_out(b):
        pltpu.make_async_copy(rows_v.at[pl.ds(b * C, C)],
                              inter_hbm.at[pl.ds(0, C)], sem_out).wait()

    def group_body(g, carry):
        c0 = start + g * Q
        c0f = jnp.minimum(c0, num_chunks - Q)
        pltpu.sync_copy(idx_hbm.at[:, pl.ds(c0f * CS, Q * CS)], idx_v)
        for q in range(Q):
            k = c0 + q

            @pl.when(k < start + cnt)
            def _():
                b = (g * Q + q) & 1
                fire_gathers(k, c0f, b)
                drain_gathers(c0f, b)

                @pl.when(g * Q + q >= 1)
                def _():
                    drain_out(1 - b)

                fire_out(k, b)

        return carry

    lax.fori_loop(0, n_groups, group_body, 0)
    drain_out(0)


def _addpos_kernel(F, D, CS, num_chunks, NC, NW,
                   in_hbm, pos_hbm, out_hbm, rows_v, buf_v, pos_v,
                   sem_in, sem_out):
    w = lax.axis_index("s") * NC + lax.axis_index("c")
    R = F * CS * D // 128
    FD = F * D
    iota_v = lax.broadcasted_iota(jnp.int32, (D,), 0)
    row_c = lax.shift_right_logical(iota_v, 3)
    dvecs = [(iota_v + j) & 15 for j in range(D)]
    lanevs = [(iota_v & 7) * D + dvecs[j] for j in range(D)]
    pltpu.sync_copy(pos_hbm, pos_v)
    n_mine = (num_chunks - 1 - w) // NW + 1

    pltpu.async_copy(in_hbm.at[pl.ds(w * R, R)],
                     rows_v.at[pl.ds(0, R)], sem_in)

    def chunk_body(k, carry):
        b = k & 1
        sb = w + k * NW
        pltpu.make_async_copy(in_hbm.at[pl.ds(0, R)],
                              rows_v.at[pl.ds(b * R, R)], sem_in).wait()

        @pl.when(k + 1 < n_mine)
        def _():
            pltpu.async_copy(in_hbm.at[pl.ds((sb + NW) * R, R)],
                             rows_v.at[pl.ds((1 - b) * R, R)], sem_in)

        @pl.when(k >= 1)
        def _():
            pltpu.make_async_copy(buf_v.at[pl.ds(0, FD)],
                                  out_hbm.at[:, pl.ds(0, CS)],
                                  sem_out).wait()

        rbase = b * R
        bbase = b * FD

        @functools.partial(plsc.parallel_loop, 0, F, unroll=2)
        def f_body(f):
            pv = pos_v[f >> 3, pl.ds((f & 7) * D, D)]
            rowvs = [row_c + (rbase + f * D + 2 * kk)
                     for kk in range(CS // D)]
            fD = bbase + f * D
            for kk in range(CS // D):
                colv = iota_v + (kk * D)
                for j in range(D):
                    vals = plsc.load_gather(rows_v, [rowvs[kk], lanevs[j]])
                    plsc.store_scatter(buf_v, [dvecs[j] + fD, colv],
                                       vals + pv)
        pltpu.async_copy(buf_v.at[pl.ds(bbase, FD)],
                         out_hbm.at[:, pl.ds(sb * CS, CS)], sem_out)
        return carry

    lax.fori_loop(0, n_mine, chunk_body, 0)
    pltpu.make_async_copy(buf_v.at[pl.ds(0, FD)],
                          out_hbm.at[:, pl.ds(0, CS)], sem_out).wait()


def kernel(inputs, token_table, pos_table):
    S, F = inputs.shape
    V, D = token_table.shape
    assert D == 16 and F <= 16
    CS = 128
    num_chunks = S // CS
    s_main = num_chunks * CS
    C = F * CS

    info = plsc.get_sparse_core_info()
    NC, NS = info.num_cores, info.num_subcores
    NW = NC * NS

    pos_block = pos_table[S:, F:]
    pos_flat = jnp.broadcast_to(pos_block, (F, D)).reshape(-1)
    pos2 = jnp.pad(pos_flat, (0, 256 - F * D)).reshape(2, 128)

    idx_t = inputs.T

    mesh = plsc.VectorSubcoreMesh(core_axis_name="c", subcore_axis_name="s")

    Q = 1
    gather_body = functools.partial(_gather_kernel, F, CS, Q, num_chunks,
                                    NC, NW)
    inter = pl.kernel(
        gather_body,
        mesh=mesh,
        compiler_params=pltpu.CompilerParams(use_tc_tiling_on_sc=False,
                                             needs_layout_passes=False),
        out_type=jax.ShapeDtypeStruct((num_chunks * C, D), jnp.float32),
        scratch_types=[
            pltpu.VMEM((F, Q * CS), jnp.int32),
            pltpu.VMEM((2 * C, D), jnp.float32),
            pltpu.SemaphoreType.DMA,
            pltpu.SemaphoreType.DMA,
        ],
    )(idx_t, token_table)

    in128 = inter.reshape(num_chunks * C * D // 128, 128)

    addpos_body = functools.partial(_addpos_kernel, F, D, CS, num_chunks,
                                    NC, NW)
    out_t = pl.kernel(
        addpos_body,
        mesh=mesh,
        compiler_params=pltpu.CompilerParams(needs_layout_passes=False),
        out_type=jax.ShapeDtypeStruct((F * D, S), jnp.float32),
        scratch_types=[
            pltpu.VMEM((2 * C * D // 128, 128), jnp.float32),
            pltpu.VMEM((2 * F * D, CS), jnp.float32),
            pltpu.VMEM((2, 128), jnp.float32),
            pltpu.SemaphoreType.DMA,
            pltpu.SemaphoreType.DMA,
        ],
    )(in128, pos2)

    out = out_t.reshape(F, D, S).transpose(2, 0, 1)
    if s_main < S:
        tail = jnp.take(token_table, inputs[s_main:], axis=0)
        tail = tail + jnp.broadcast_to(pos_block, (F, D))[None]
        out = lax.dynamic_update_slice(out, tail, (s_main, 0, 0))
    return out
